# Initial kernel scaffold; baseline (speedup 1.0000x reference)
import jax
import jax.numpy as jnp
from jax.experimental import pallas as pl


def kernel(user_table, item_table, edge_src, edge_dst, edge_weight, user_indices, item_indices):
    raise NotImplementedError("write your pallas kernel here")


# speedup vs baseline: 6.6436x; 6.6436x over previous
import jax
import jax.numpy as jnp
from jax import lax
from jax.experimental import pallas as pl
from jax.experimental.pallas import tpu as pltpu
from jax.experimental.pallas import tpu_sc as plsc

N_USERS = 50000
N_TOTAL = 100000
EMB = 64
N_EDGES = 1000000
N_LAYERS = 3
BATCH = 4096

NC = 2
NS = 16
L = 16

W = 512
K = W // 128
NWIN = 124
E_PAD = NS * NWIN * W

N_PAD = 100096
ROWS_PER_TILE = N_PAD // NS
FCH = 184
NFL = ROWS_PER_TILE // FCH

B2 = 2 * BATCH
BPT = B2 // NS
BK = BPT // 128


def _gcn_body(table_hbm, esrc_hbm, edst_hbm, ew_hbm, bidx_hbm,
              out_hbm, cur_hbm,
              A, srcb, dstb, wb, gix, rows, zb, fb, bixb, tgix, brows, bacc,
              sem1, semg):
    c = lax.axis_index("c")
    s = lax.axis_index("s")

    for k in range(BK):
        pltpu.sync_copy(bidx_hbm.at[pl.ds(s * BPT + k * 128, 128)], bixb.at[k])

    @pl.loop(0, FCH)
    def _zero_zb(r):
        zb[r, :] = jnp.zeros((L,), jnp.float32)

    def do_pass(layer, p):
        cc = 2 * c + p

        @pl.loop(0, NFL)
        def _zero(k):
            pltpu.sync_copy(zb, A.at[pl.ds(s * ROWS_PER_TILE + k * FCH, FCH)])

        plsc.subcore_barrier()

        if layer == 0:
            src_ref = table_hbm
            mul = jnp.int32(4)
            off = cc
        else:
            src_ref = cur_hbm
            mul = jnp.int32(1)
            off = ((layer - 1) * 4 + cc) * N_PAD

        @pl.loop(0, NWIN)
        def _win(wi):
            base = (s * NWIN + wi) * W
            cps = [pltpu.async_copy(esrc_hbm.at[pl.ds(base, W)], srcb, sem1),
                   pltpu.async_copy(ew_hbm.at[pl.ds(base, W)], wb, sem1)]
            for k in range(K):
                cps.append(pltpu.async_copy(
                    edst_hbm.at[pl.ds(base + k * 128, 128)], dstb.at[k], sem1))
            for cp in cps:
                cp.wait()

            @pl.loop(0, K)
            def _gi(k):
                for j in range(8):
                    sb = srcb[pl.ds(k * 128 + j * L, L)]
                    gix[k, pl.ds(j * L, L)] = sb * mul + off

            hs = [pltpu.async_copy(src_ref.at[gix.at[k]],
                                   rows.at[pl.ds(k * 128, 128)], semg)
                  for k in range(K)]
            for h in hs:
                h.wait()

            @pl.loop(0, W // L)
            def _scale(g):
                wv = wb[pl.ds(g * L, L)]
                for i in range(L):
                    splat = wv.at[jnp.full((L,), i, jnp.int32)].get(
                        mode="promise_in_bounds")
                    rows[g * L + i, :] = rows[g * L + i, :] * splat

            for k in range(K):
                pltpu.sync_copy(rows.at[pl.ds(k * 128, 128)],
                                A.at[dstb.at[k]], add=True)

        plsc.subcore_barrier()

        @pl.loop(0, NFL)
        def _flush(k):
            rb = s * ROWS_PER_TILE + k * FCH
            pltpu.sync_copy(A.at[pl.ds(rb, FCH)], fb)
            pltpu.sync_copy(
                fb, cur_hbm.at[pl.ds((layer * 4 + cc) * N_PAD + rb, FCH)])

        plsc.subcore_barrier()

    for layer in range(N_LAYERS):
        for p in range(2):
            do_pass(layer, p)

    for p in range(2):
        cc = 2 * c + p
        for h in range(2):

            @pl.loop(0, BPT // 2)
            def _zacc(r):
                bacc[r, :] = jnp.zeros((L,), jnp.float32)

            for src_i in range(1 + N_LAYERS):

                @pl.loop(0, 2)
                def _ti(k):
                    for j in range(8):
                        bb = bixb[2 * h + k, pl.ds(j * L, L)]
                        if src_i == 0:
                            gi = bb * 4 + cc
                        else:
                            gi = bb + ((src_i - 1) * 4 + cc) * N_PAD
                        tgix[k, pl.ds(j * L, L)] = gi

                sref = table_hbm if src_i == 0 else cur_hbm
                hs = [pltpu.async_copy(sref.at[tgix.at[k]],
                                       brows.at[pl.ds(k * 128, 128)], semg)
                      for k in range(2)]
                for hh in hs:
                    hh.wait()

                @pl.loop(0, BPT // 2)
                def _bacc(r):
                    bacc[r, :] = bacc[r, :] + brows[r, :]

            @pl.loop(0, BPT // 2)
            def _fin(r):
                brows[r, :] = bacc[r, :] * (1.0 / (N_LAYERS + 1))

            pltpu.sync_copy(
                brows,
                out_hbm.at[pl.ds(cc * B2 + s * BPT + h * (BPT // 2),
                                 BPT // 2)])


@jax.jit
def kernel(user_table, item_table, edge_src, edge_dst, edge_weight,
           user_indices, item_indices):
    table = jnp.concatenate([user_table, item_table], axis=0)
    table_v = table.reshape(N_TOTAL * 4, L)

    pad = E_PAD - N_EDGES
    pidx = jnp.arange(pad, dtype=jnp.int32) % N_TOTAL
    esrc = jnp.concatenate([edge_src, pidx])
    edst = jnp.concatenate([edge_dst, pidx])
    ew = jnp.concatenate([edge_weight, jnp.zeros((pad,), jnp.float32)])
    bidx = jnp.concatenate([user_indices, item_indices + N_USERS])

    mesh = plsc.VectorSubcoreMesh(core_axis_name="c", subcore_axis_name="s")
    run = pl.kernel(
        _gcn_body,
        out_type=[
            jax.ShapeDtypeStruct((4 * B2, L), jnp.float32),
            jax.ShapeDtypeStruct((N_LAYERS * 4 * N_PAD, L), jnp.float32),
        ],
        mesh=mesh,
        compiler_params=pltpu.CompilerParams(use_tc_tiling_on_sc=False),
        scratch_types=[
            pltpu.VMEM_SHARED((N_PAD, L), jnp.float32),
            pltpu.VMEM((W,), jnp.int32),
            pltpu.VMEM((K, 128), jnp.int32),
            pltpu.VMEM((W,), jnp.float32),
            pltpu.VMEM((K, 128), jnp.int32),
            pltpu.VMEM((W, L), jnp.float32),
            pltpu.VMEM((FCH, L), jnp.float32),
            pltpu.VMEM((FCH, L), jnp.float32),
            pltpu.VMEM((BK, 128), jnp.int32),
            pltpu.VMEM((2, 128), jnp.int32),
            pltpu.VMEM((BPT // 2, L), jnp.float32),
            pltpu.VMEM((BPT // 2, L), jnp.float32),
            pltpu.SemaphoreType.DMA,
            pltpu.SemaphoreType.DMA,
        ],
    )
    out, _ = run(table_v, esrc, edst, ew, bidx)
    out = out.reshape(4, B2, L).transpose(1, 0, 2).reshape(B2, EMB)
    return out[:BATCH], out[BATCH:]


# speedup vs baseline: 13.0489x; 1.9641x over previous
import jax
import jax.numpy as jnp
from jax import lax
from jax.experimental import pallas as pl
from jax.experimental.pallas import tpu as pltpu
from jax.experimental.pallas import tpu_sc as plsc

N_USERS = 50000
N_TOTAL = 100000
EMB = 64
N_EDGES = 1000000
N_LAYERS = 3
BATCH = 4096

NC = 2
NS = 16
L = 16

W = 512
K = W // 128
NWIN = 124
E_PAD = NS * NWIN * W

N_PAD = 100096
ROWS_PER_TILE = N_PAD // NS
FCH = 136
NFL = ROWS_PER_TILE // FCH

B2 = 2 * BATCH
BPT = B2 // NS
BK = BPT // 128


def _gcn_body(table_hbm, esrc_hbm, edst_hbm, ew_hbm, bidx_hbm,
              out_hbm, cur_hbm,
              A, srcb, dstb, sidx, wb, gix, rows, zfb, bixb, tgix, brows,
              bacc, sem_l0, sem_l1, sem_g0, sem_g1, sem_s0, sem_s1):
    c = lax.axis_index("c")
    s = lax.axis_index("s")
    sem_l = (sem_l0, sem_l1)
    sem_g = (sem_g0, sem_g1)
    sem_s = (sem_s0, sem_s1)

    for k in range(BK):
        pltpu.sync_copy(bidx_hbm.at[pl.ds(s * BPT + k * 128, 128)], bixb.at[k])

    def issue_linear(wi, b):
        base = (s * NWIN + wi) * W
        pltpu.async_copy(esrc_hbm.at[pl.ds(base, W)], srcb.at[b], sem_l[b])
        pltpu.async_copy(ew_hbm.at[pl.ds(base, W)], wb.at[b], sem_l[b])
        pltpu.async_copy(edst_hbm.at[pl.ds(base // 128, K)], dstb.at[b],
                         sem_l[b])

    def wait_linear(b):
        pltpu.make_async_copy(esrc_hbm.at[pl.ds(0, W)], srcb.at[b],
                              sem_l[b]).wait()
        pltpu.make_async_copy(ew_hbm.at[pl.ds(0, W)], wb.at[b],
                              sem_l[b]).wait()
        pltpu.make_async_copy(edst_hbm.at[pl.ds(0, K)], dstb.at[b],
                              sem_l[b]).wait()

    def drain_scatters(b):
        for k in range(K):
            pltpu.make_async_copy(rows.at[b].at[pl.ds(k * 128, 128)],
                                  A.at[sidx.at[b].at[k]], sem_s[b]).wait()

    def do_pass(layer, p):
        cc = 2 * c + p

        @pl.loop(0, FCH)
        def _zero_zfb(r):
            zfb[r, :] = jnp.zeros((L,), jnp.float32)

        @pl.loop(0, NFL)
        def _zero(k):
            pltpu.sync_copy(zfb, A.at[pl.ds(s * ROWS_PER_TILE + k * FCH,
                                            FCH)])

        plsc.subcore_barrier()

        if layer == 0:
            src_ref = table_hbm
            mul = jnp.int32(4)
            off = cc
        else:
            src_ref = cur_hbm
            mul = jnp.int32(1)
            off = ((layer - 1) * 4 + cc) * N_PAD

        for b in range(2):
            issue_linear(b, b)

        @pl.loop(0, NWIN, step=2)
        def _outer(wi0):
            for b in range(2):

                @pl.when(wi0 >= 2)
                def _drain_old():
                    drain_scatters(b)

                wait_linear(b)

                @pl.loop(0, K)
                def _gi(k):
                    for j in range(8):
                        sb = srcb[b, pl.ds(k * 128 + j * L, L)]
                        gix[b, k, pl.ds(j * L, L)] = sb * mul + off
                        sidx[b, k, pl.ds(j * L, L)] = \
                            dstb[b, k, pl.ds(j * L, L)]

                for k in range(K):
                    pltpu.async_copy(src_ref.at[gix.at[b].at[k]],
                                     rows.at[b].at[pl.ds(k * 128, 128)],
                                     sem_g[b])

            for b in range(2):
                for k in range(K):
                    pltpu.make_async_copy(src_ref.at[gix.at[b].at[k]],
                                          rows.at[b].at[pl.ds(k * 128, 128)],
                                          sem_g[b]).wait()

                @pl.loop(0, W // L)
                def _scale(g):
                    wv = wb[b, pl.ds(g * L, L)]
                    for i in range(L):
                        splat = wv.at[jnp.full((L,), i, jnp.int32)].get(
                            mode="promise_in_bounds")
                        rows[b, g * L + i, :] = rows[b, g * L + i, :] * splat

                for k in range(K):
                    pltpu.make_async_copy(
                        rows.at[b].at[pl.ds(k * 128, 128)],
                        A.at[sidx.at[b].at[k]],
                        sem_s[b]).start(add=True)

                @pl.when(wi0 + b + 2 < NWIN)
                def _prefetch():
                    issue_linear(wi0 + b + 2, b)

        for b in range(2):
            drain_scatters(b)

        plsc.subcore_barrier()

        @pl.loop(0, NFL)
        def _flush(k):
            rb = s * ROWS_PER_TILE + k * FCH
            pltpu.sync_copy(A.at[pl.ds(rb, FCH)], zfb)
            pltpu.sync_copy(
                zfb, cur_hbm.at[pl.ds((layer * 4 + cc) * N_PAD + rb, FCH)])

        plsc.subcore_barrier()

    for layer in range(N_LAYERS):
        for p in range(2):
            do_pass(layer, p)

    for p in range(2):
        cc = 2 * c + p
        for q in range(BK):

            @pl.loop(0, 128)
            def _zacc(r):
                bacc[r, :] = jnp.zeros((L,), jnp.float32)

            for src_i in range(1 + N_LAYERS):

                @pl.loop(0, 8)
                def _ti(j):
                    bb = bixb[q, pl.ds(j * L, L)]
                    if src_i == 0:
                        gi = bb * 4 + cc
                    else:
                        gi = bb + ((src_i - 1) * 4 + cc) * N_PAD
                    tgix[pl.ds(j * L, L)] = gi

                sref = table_hbm if src_i == 0 else cur_hbm
                pltpu.sync_copy(sref.at[tgix], brows)

                @pl.loop(0, 128)
                def _bacc(r):
                    bacc[r, :] = bacc[r, :] + brows[r, :]

            @pl.loop(0, 128)
            def _fin(r):
                brows[r, :] = bacc[r, :] * (1.0 / (N_LAYERS + 1))

            pltpu.sync_copy(
                brows, out_hbm.at[pl.ds(cc * B2 + s * BPT + q * 128, 128)])


@jax.jit
def kernel(user_table, item_table, edge_src, edge_dst, edge_weight,
           user_indices, item_indices):
    table = jnp.concatenate([user_table, item_table], axis=0)
    table_v = table.reshape(N_TOTAL * 4, L)

    pad = E_PAD - N_EDGES
    pidx = jnp.arange(pad, dtype=jnp.int32) % N_TOTAL
    esrc = jnp.concatenate([edge_src, pidx])
    edst = jnp.concatenate([edge_dst, pidx]).reshape(E_PAD // 128, 128)
    ew = jnp.concatenate([edge_weight, jnp.zeros((pad,), jnp.float32)])
    bidx = jnp.concatenate([user_indices, item_indices + N_USERS])

    mesh = plsc.VectorSubcoreMesh(core_axis_name="c", subcore_axis_name="s")
    run = pl.kernel(
        _gcn_body,
        out_type=[
            jax.ShapeDtypeStruct((4 * B2, L), jnp.float32),
            jax.ShapeDtypeStruct((N_LAYERS * 4 * N_PAD, L), jnp.float32),
        ],
        mesh=mesh,
        compiler_params=pltpu.CompilerParams(use_tc_tiling_on_sc=False),
        scratch_types=[
            pltpu.VMEM_SHARED((N_PAD, L), jnp.float32),
            pltpu.VMEM((2, W), jnp.int32),
            pltpu.VMEM((2, K, 128), jnp.int32),
            pltpu.VMEM((2, K, 128), jnp.int32),
            pltpu.VMEM((2, W), jnp.float32),
            pltpu.VMEM((2, K, 128), jnp.int32),
            pltpu.VMEM((2, W, L), jnp.float32),
            pltpu.VMEM((FCH, L), jnp.float32),
            pltpu.VMEM((BK, 128), jnp.int32),
            pltpu.VMEM((128,), jnp.int32),
            pltpu.VMEM((128, L), jnp.float32),
            pltpu.VMEM((128, L), jnp.float32),
            pltpu.SemaphoreType.DMA,
            pltpu.SemaphoreType.DMA,
            pltpu.SemaphoreType.DMA,
            pltpu.SemaphoreType.DMA,
            pltpu.SemaphoreType.DMA,
            pltpu.SemaphoreType.DMA,
        ],
    )
    out, _ = run(table_v, esrc, edst, ew, bidx)
    out = out.reshape(4, B2, L).transpose(1, 0, 2).reshape(B2, EMB)
    return out[:BATCH], out[BATCH:]


# speedup vs baseline: 14.2230x; 1.0900x over previous
import jax
import jax.numpy as jnp
from jax import lax
from jax.experimental import pallas as pl
from jax.experimental.pallas import tpu as pltpu
from jax.experimental.pallas import tpu_sc as plsc

N_USERS = 50000
N_TOTAL = 100000
EMB = 64
N_EDGES = 1000000
N_LAYERS = 3
BATCH = 4096

NC = 2
NS = 16
L = 16

W = 512
K = W // 128
NWIN = 124
E_PAD = NS * NWIN * W

N_PAD = 100096
ROWS_PER_TILE = N_PAD // NS
FCH = 136
NFL = ROWS_PER_TILE // FCH

B2 = 2 * BATCH
BPT = B2 // NS
BK = BPT // 128


def _gcn_body(table_hbm, esrc_hbm, edst_hbm, ew_hbm, bidx_hbm,
              out_hbm, cur_hbm,
              A, srcb, dstb, sidx, wb, gix, rows, zfb, bixb, tgix, brows,
              bacc, sem_l0, sem_l1, sem_g0, sem_g1, sem_s0, sem_s1):
    c = lax.axis_index("c")
    s = lax.axis_index("s")
    sem_l = (sem_l0, sem_l1)
    sem_g = (sem_g0, sem_g1)
    sem_s = (sem_s0, sem_s1)

    for k in range(BK):
        pltpu.sync_copy(bidx_hbm.at[pl.ds(s * BPT + k * 128, 128)], bixb.at[k])

    @pl.loop(0, FCH)
    def _fill_zfb(r):
        zfb[r, :] = jnp.zeros((L,), jnp.float32)

    def issue_linear(wi, b):
        base = (s * NWIN + wi) * W
        pltpu.async_copy(esrc_hbm.at[pl.ds(base, W)], srcb.at[b], sem_l[b])
        pltpu.async_copy(ew_hbm.at[pl.ds(base, W)], wb.at[b], sem_l[b])
        pltpu.async_copy(edst_hbm.at[pl.ds(base // 128, K)], dstb.at[b],
                         sem_l[b])

    def wait_linear(b):
        pltpu.make_async_copy(esrc_hbm.at[pl.ds(0, W)], srcb.at[b],
                              sem_l[b]).wait()
        pltpu.make_async_copy(ew_hbm.at[pl.ds(0, W)], wb.at[b],
                              sem_l[b]).wait()
        pltpu.make_async_copy(edst_hbm.at[pl.ds(0, K)], dstb.at[b],
                              sem_l[b]).wait()

    def drain_scatters(b):
        for k in range(K):
            pltpu.make_async_copy(rows.at[b].at[pl.ds(k * 128, 128)],
                                  A.at[sidx.at[b].at[k]], sem_s[b]).wait()

    def do_pass(layer, p):
        cc = 2 * c + p

        if layer == 0 and p == 0:
            @pl.loop(0, NFL)
            def _zero_fire(k):
                pltpu.async_copy(
                    zfb, A.at[pl.ds(s * ROWS_PER_TILE + k * FCH, FCH)],
                    sem_g[0])

            @pl.loop(0, NFL)
            def _zero_drain(k):
                pltpu.make_async_copy(
                    zfb, A.at[pl.ds(s * ROWS_PER_TILE + k * FCH, FCH)],
                    sem_g[0]).wait()

        plsc.subcore_barrier()

        if layer == 0:
            src_ref = table_hbm
            mul = jnp.int32(4)
            off = cc
        else:
            src_ref = cur_hbm
            mul = jnp.int32(1)
            off = ((layer - 1) * 4 + cc) * N_PAD

        for b in range(2):
            issue_linear(b, b)

        @pl.loop(0, NWIN, step=2)
        def _outer(wi0):
            for b in range(2):

                @pl.when(wi0 >= 2)
                def _drain_old():
                    drain_scatters(b)

                wait_linear(b)

                @pl.loop(0, K)
                def _gi(k):
                    for j in range(8):
                        sb = srcb[b, pl.ds(k * 128 + j * L, L)]
                        gix[b, k, pl.ds(j * L, L)] = sb * mul + off
                        sidx[b, k, pl.ds(j * L, L)] = \
                            dstb[b, k, pl.ds(j * L, L)]

                for k in range(K):
                    pltpu.async_copy(src_ref.at[gix.at[b].at[k]],
                                     rows.at[b].at[pl.ds(k * 128, 128)],
                                     sem_g[b])

            for b in range(2):
                for k in range(K):
                    pltpu.make_async_copy(src_ref.at[gix.at[b].at[k]],
                                          rows.at[b].at[pl.ds(k * 128, 128)],
                                          sem_g[b]).wait()

                @plsc.parallel_loop(0, W // L, unroll=2)
                def _scale(g):
                    wv = wb[b, pl.ds(g * L, L)]
                    for i in range(L):
                        splat = wv.at[jnp.full((L,), i, jnp.int32)].get(
                            mode="promise_in_bounds")
                        rows[b, g * L + i, :] = rows[b, g * L + i, :] * splat

                for k in range(K):
                    pltpu.make_async_copy(
                        rows.at[b].at[pl.ds(k * 128, 128)],
                        A.at[sidx.at[b].at[k]],
                        sem_s[b]).start(add=True)

                @pl.when(wi0 + b + 2 < NWIN)
                def _prefetch():
                    issue_linear(wi0 + b + 2, b)

        for b in range(2):
            drain_scatters(b)

        plsc.subcore_barrier()

        hoff = (layer * 4 + cc) * N_PAD

        @pl.loop(0, NFL)
        def _flush_fire(k):
            rb = s * ROWS_PER_TILE + k * FCH
            pltpu.async_copy(A.at[pl.ds(rb, FCH)],
                             cur_hbm.at[pl.ds(hoff + rb, FCH)], sem_g[1])

        @pl.loop(0, NFL)
        def _flush_drain(k):
            rb = s * ROWS_PER_TILE + k * FCH
            pltpu.make_async_copy(A.at[pl.ds(rb, FCH)],
                                  cur_hbm.at[pl.ds(hoff + rb, FCH)],
                                  sem_g[1]).wait()

        if not (layer == N_LAYERS - 1 and p == 1):
            @pl.loop(0, NFL)
            def _rz_fire(k):
                pltpu.async_copy(
                    zfb, A.at[pl.ds(s * ROWS_PER_TILE + k * FCH, FCH)],
                    sem_g[0])

            @pl.loop(0, NFL)
            def _rz_drain(k):
                pltpu.make_async_copy(
                    zfb, A.at[pl.ds(s * ROWS_PER_TILE + k * FCH, FCH)],
                    sem_g[0]).wait()

    for layer in range(N_LAYERS):
        for p in range(2):
            do_pass(layer, p)

    for p in range(2):
        cc = 2 * c + p
        for q in range(BK):

            @pl.loop(0, 128)
            def _zacc(r):
                bacc[r, :] = jnp.zeros((L,), jnp.float32)

            for src_i in range(1 + N_LAYERS):

                @pl.loop(0, 8)
                def _ti(j):
                    bb = bixb[q, pl.ds(j * L, L)]
                    if src_i == 0:
                        gi = bb * 4 + cc
                    else:
                        gi = bb + ((src_i - 1) * 4 + cc) * N_PAD
                    tgix[pl.ds(j * L, L)] = gi

                sref = table_hbm if src_i == 0 else cur_hbm
                pltpu.sync_copy(sref.at[tgix], brows)

                @pl.loop(0, 128)
                def _bacc(r):
                    bacc[r, :] = bacc[r, :] + brows[r, :]

            @pl.loop(0, 128)
            def _fin(r):
                brows[r, :] = bacc[r, :] * (1.0 / (N_LAYERS + 1))

            pltpu.sync_copy(
                brows, out_hbm.at[pl.ds(cc * B2 + s * BPT + q * 128, 128)])


@jax.jit
def kernel(user_table, item_table, edge_src, edge_dst, edge_weight,
           user_indices, item_indices):
    table = jnp.concatenate([user_table, item_table], axis=0)
    table_v = table.reshape(N_TOTAL * 4, L)

    pad = E_PAD - N_EDGES
    pidx = jnp.arange(pad, dtype=jnp.int32) % N_TOTAL
    esrc = jnp.concatenate([edge_src, pidx])
    edst = jnp.concatenate([edge_dst, pidx]).reshape(E_PAD // 128, 128)
    ew = jnp.concatenate([edge_weight, jnp.zeros((pad,), jnp.float32)])
    bidx = jnp.concatenate([user_indices, item_indices + N_USERS])

    mesh = plsc.VectorSubcoreMesh(core_axis_name="c", subcore_axis_name="s")
    run = pl.kernel(
        _gcn_body,
        out_type=[
            jax.ShapeDtypeStruct((4 * B2, L), jnp.float32),
            jax.ShapeDtypeStruct((N_LAYERS * 4 * N_PAD, L), jnp.float32),
        ],
        mesh=mesh,
        compiler_params=pltpu.CompilerParams(use_tc_tiling_on_sc=False),
        scratch_types=[
            pltpu.VMEM_SHARED((N_PAD, L), jnp.float32),
            pltpu.VMEM((2, W), jnp.int32),
            pltpu.VMEM((2, K, 128), jnp.int32),
            pltpu.VMEM((2, K, 128), jnp.int32),
            pltpu.VMEM((2, W), jnp.float32),
            pltpu.VMEM((2, K, 128), jnp.int32),
            pltpu.VMEM((2, W, L), jnp.float32),
            pltpu.VMEM((FCH, L), jnp.float32),
            pltpu.VMEM((BK, 128), jnp.int32),
            pltpu.VMEM((128,), jnp.int32),
            pltpu.VMEM((128, L), jnp.float32),
            pltpu.VMEM((128, L), jnp.float32),
            pltpu.SemaphoreType.DMA,
            pltpu.SemaphoreType.DMA,
            pltpu.SemaphoreType.DMA,
            pltpu.SemaphoreType.DMA,
            pltpu.SemaphoreType.DMA,
            pltpu.SemaphoreType.DMA,
        ],
    )
    out, _ = run(table_v, esrc, edst, ew, bidx)
    out = out.reshape(4, B2, L).transpose(1, 0, 2).reshape(B2, EMB)
    return out[:BATCH], out[BATCH:]
